# baseline (speedup 1.0000x reference)
import functools

import jax
import jax.numpy as jnp
from jax import lax
from jax.experimental import pallas as pl
from jax.experimental.pallas import tpu as pltpu
from jax.experimental.pallas import tpu_sc as plsc

_PREC = lax.Precision.DEFAULT


def _dot(a, b):
    return jnp.dot(a, b, preferred_element_type=jnp.float32, precision=_PREC)

_SC_CORES = 2
_SC_SUBCORES = 16
_SC_LANES = 16


def _router_sc(logitsT):
    NE, TOK = logitsT.shape
    NW = _SC_CORES * _SC_SUBCORES
    TPW = TOK // NW
    mesh = plsc.VectorSubcoreMesh(
        core_axis_name="c", subcore_axis_name="s",
        num_cores=_SC_CORES, num_subcores=_SC_SUBCORES)

    @functools.partial(
        pl.kernel, mesh=mesh,
        out_type=[jax.ShapeDtypeStruct((2, TOK), jnp.float32),
                  jax.ShapeDtypeStruct((2, TOK), jnp.int32)],
        scratch_types=[pltpu.VMEM((NE, TPW), jnp.float32),
                       pltpu.VMEM((2, TPW), jnp.float32),
                       pltpu.VMEM((2, TPW), jnp.int32)],
    )
    def run(lg_hbm, w_hbm, i_hbm, lg_v, w_v, i_v):
        wid = lax.axis_index("s") * _SC_CORES + lax.axis_index("c")
        base = wid * TPW
        pltpu.sync_copy(lg_hbm.at[:, pl.ds(base, TPW)], lg_v)
        for g in range(TPW // _SC_LANES):
            sl = pl.ds(g * _SC_LANES, _SC_LANES)
            best1 = lg_v[0, sl]
            i1 = jnp.zeros((_SC_LANES,), jnp.int32)
            best2 = jnp.full((_SC_LANES,), -1e30, jnp.float32)
            i2 = jnp.zeros((_SC_LANES,), jnp.int32)
            for e in range(1, NE):
                v = lg_v[e, sl]
                gt1 = v > best1
                gt2 = v > best2
                i2 = jnp.where(gt1, i1, jnp.where(gt2, e, i2))
                best2 = jnp.where(gt1, best1, jnp.where(gt2, v, best2))
                i1 = jnp.where(gt1, e, i1)
                best1 = jnp.where(gt1, v, best1)
            w1 = 1.0 / (1.0 + jnp.exp(best2 - best1))
            w_v[0, sl] = w1
            w_v[1, sl] = 1.0 - w1
            i_v[0, sl] = i1
            i_v[1, sl] = i2
        pltpu.sync_copy(w_v, w_hbm.at[:, pl.ds(base, TPW)])
        pltpu.sync_copy(i_v, i_hbm.at[:, pl.ds(base, TPW)])

    return run(logitsT)


def _sigmoid(v):
    return lax.logistic(v)


def _silu(v):
    return v * _sigmoid(v)


def _softplus(v):
    return jnp.logaddexp(v, 0.0)


def _mamba_body(C, DI, DS, DCONV, NE,
                x_ref, winT_ref, cwT_ref, cb_ref, xpdtT_ref, xpBT_ref,
                xpCT_ref, dtT_ref, dtb_ref, alogT_ref, d_ref, opT_ref,
                rwT_ref, rb_ref,
                lg_ref,
                carry_ref, h_ref, dsc_ref, u_ref, g_ref, ud_ref,
                bm_ref, cm_ref, yg_ref):
    c = pl.program_id(1)

    @pl.when(c == 0)
    def _init():
        carry_ref[...] = jnp.zeros_like(carry_ref)
        h_ref[...] = jnp.zeros_like(h_ref)

    xb = x_ref[0]
    xz = _dot(xb, winT_ref[...])
    xpart = xz[:, :DI]
    res_ = xz[:, DI:]

    ext = jnp.concatenate([carry_ref[...], xpart], axis=0)
    carry_ref[...] = xpart[C - (DCONV - 1):, :]
    conv_out = jnp.broadcast_to(cb_ref[...], (C, DI))
    for k in range(DCONV):
        conv_out = conv_out + cwT_ref[k:k + 1, :] * ext[k:k + C, :]
    u = _silu(conv_out)

    dt = _dot(u, xpdtT_ref[...])
    bm_ref[...] = _dot(u, xpBT_ref[...])
    cm_ref[...] = _dot(u, xpCT_ref[...])
    z = _dot(dt, dtT_ref[...]) + dtb_ref[...]
    delta = _softplus(z)
    dsc_ref[...] = delta
    u_ref[...] = u
    g_ref[...] = _silu(res_)
    ud_ref[...] = u * d_ref[...]

    AT = -jnp.exp(alogT_ref[...])
    eye = (lax.broadcasted_iota(jnp.int32, (DS, DS), 0)
           == lax.broadcasted_iota(jnp.int32, (DS, DS), 1)).astype(jnp.float32)

    def substep(t, h):
        drow = dsc_ref[pl.ds(t, 1), :]
        urow = u_ref[pl.ds(t, 1), :]
        brow = bm_ref[pl.ds(t, 1), :]
        crow = cm_ref[pl.ds(t, 1), :]
        E = jnp.exp(drow * AT)
        bcol = jnp.sum(jnp.broadcast_to(brow, (DS, DS)) * eye, axis=1,
                       keepdims=True)
        ccol = jnp.sum(jnp.broadcast_to(crow, (DS, DS)) * eye, axis=1,
                       keepdims=True)
        h = E * h + (drow * bcol) * urow
        yg_ref[pl.ds(t, 1), :] = jnp.sum(h * ccol, axis=0, keepdims=True)
        return h

    def step(i, h):
        t = i * 2
        h = substep(t, h)
        return substep(t + 1, h)

    h_ref[...] = lax.fori_loop(0, C // 2, step, h_ref[...])

    yg = (yg_ref[...] + ud_ref[...]) * g_ref[...]
    ctx = _dot(yg, opT_ref[...])
    lg_ref[0] = _dot(ctx, rwT_ref[...]) + rb_ref[...]


def kernel(x, in_proj_w, conv_w, conv_b, x_proj_w, dt_proj_w, dt_proj_b,
           A_log, D, out_proj_w, router_w, router_b):
    B, L, DM = x.shape
    DI, DCONV = conv_w.shape
    DTR = dt_proj_w.shape[1]
    DS = A_log.shape[1]
    NE = router_w.shape[0]
    C = min(256, L)
    NC = L // C

    winT = in_proj_w.T
    cwT = conv_w.T
    xpdtT = x_proj_w[:DTR].T
    xpBT = x_proj_w[DTR:DTR + DS].T
    xpCT = x_proj_w[DTR + DS:].T
    dtT = dt_proj_w.T
    alogT = A_log.T
    opT = out_proj_w.T
    rwT = router_w.T
    cb = conv_b.reshape(1, DI)
    dtb = dt_proj_b.reshape(1, DI)
    dd = D.reshape(1, DI)
    rb = router_b.reshape(1, NE)

    full = lambda a: pl.BlockSpec(a.shape, lambda b, c: (0,) * a.ndim)
    body = functools.partial(_mamba_body, C, DI, DS, DCONV, NE)
    logits = pl.pallas_call(
        body,
        grid=(B, NC),
        in_specs=[
            pl.BlockSpec((1, C, DM), lambda b, c: (b, c, 0)),
            full(winT), full(cwT), full(cb), full(xpdtT), full(xpBT),
            full(xpCT), full(dtT), full(dtb), full(alogT), full(dd),
            full(opT), full(rwT), full(rb),
        ],
        out_specs=pl.BlockSpec((1, C, NE), lambda b, c: (b, c, 0)),
        out_shape=jax.ShapeDtypeStruct((B, L, NE), jnp.float32),
        scratch_shapes=[
            pltpu.VMEM((DCONV - 1, DI), jnp.float32),
            pltpu.VMEM((DS, DI), jnp.float32),
            pltpu.VMEM((C, DI), jnp.float32),
            pltpu.VMEM((C, DI), jnp.float32),
            pltpu.VMEM((C, DI), jnp.float32),
            pltpu.VMEM((C, DI), jnp.float32),
            pltpu.VMEM((C, DS), jnp.float32),
            pltpu.VMEM((C, DS), jnp.float32),
            pltpu.VMEM((C, DI), jnp.float32),
        ],
    )(x, winT, cwT, cb, xpdtT, xpBT, xpCT, dtT, dtb, alogT, dd, opT, rwT, rb)

    logitsT = logits.reshape(B * L, NE).T
    wT, iT = _router_sc(logitsT)
    weights = wT.T.reshape(B, L, 2)
    idx = iT.T.reshape(B, L, 2)
    return (weights, idx)


# speedup vs baseline: 1.0833x; 1.0833x over previous
import functools

import jax
import jax.numpy as jnp
from jax import lax
from jax.experimental import pallas as pl
from jax.experimental.pallas import tpu as pltpu
from jax.experimental.pallas import tpu_sc as plsc

_PREC = lax.Precision.DEFAULT


def _dot(a, b):
    return jnp.dot(a, b, preferred_element_type=jnp.float32, precision=_PREC)

_SC_CORES = 2
_SC_SUBCORES = 16
_SC_LANES = 16


def _router_sc(logitsT):
    NE, TOK = logitsT.shape
    NW = _SC_CORES * _SC_SUBCORES
    TPW = TOK // NW
    mesh = plsc.VectorSubcoreMesh(
        core_axis_name="c", subcore_axis_name="s",
        num_cores=_SC_CORES, num_subcores=_SC_SUBCORES)

    @functools.partial(
        pl.kernel, mesh=mesh,
        out_type=[jax.ShapeDtypeStruct((2, TOK), jnp.float32),
                  jax.ShapeDtypeStruct((2, TOK), jnp.int32)],
        scratch_types=[pltpu.VMEM((NE, TPW), jnp.float32),
                       pltpu.VMEM((2, TPW), jnp.float32),
                       pltpu.VMEM((2, TPW), jnp.int32)],
    )
    def run(lg_hbm, w_hbm, i_hbm, lg_v, w_v, i_v):
        wid = lax.axis_index("s") * _SC_CORES + lax.axis_index("c")
        base = wid * TPW
        pltpu.sync_copy(lg_hbm.at[:, pl.ds(base, TPW)], lg_v)
        for g in range(TPW // _SC_LANES):
            sl = pl.ds(g * _SC_LANES, _SC_LANES)
            best1 = lg_v[0, sl]
            i1 = jnp.zeros((_SC_LANES,), jnp.int32)
            best2 = jnp.full((_SC_LANES,), -1e30, jnp.float32)
            i2 = jnp.zeros((_SC_LANES,), jnp.int32)
            for e in range(1, NE):
                v = lg_v[e, sl]
                gt1 = v > best1
                gt2 = v > best2
                i2 = jnp.where(gt1, i1, jnp.where(gt2, e, i2))
                best2 = jnp.where(gt1, best1, jnp.where(gt2, v, best2))
                i1 = jnp.where(gt1, e, i1)
                best1 = jnp.where(gt1, v, best1)
            w1 = 1.0 / (1.0 + jnp.exp(best2 - best1))
            w_v[0, sl] = w1
            w_v[1, sl] = 1.0 - w1
            i_v[0, sl] = i1
            i_v[1, sl] = i2
        pltpu.sync_copy(w_v, w_hbm.at[:, pl.ds(base, TPW)])
        pltpu.sync_copy(i_v, i_hbm.at[:, pl.ds(base, TPW)])

    return run(logitsT)


def _sigmoid(v):
    return lax.logistic(v)


def _silu(v):
    return v * _sigmoid(v)


def _softplus(v):
    return jnp.logaddexp(v, 0.0)


def _mamba_body(C, DI, DS, DCONV, NE,
                x_ref, winT_ref, cwT_ref, cb_ref, xpdtT_ref, xpBT_ref,
                xpCT_ref, dtT_ref, dtb_ref, alogT_ref, d_ref, opT_ref,
                rwT_ref, rb_ref,
                lg_ref,
                carry_ref, h_ref, dsc_ref, u_ref, g_ref, ud_ref,
                bm_ref, cm_ref, yg_ref):
    c = pl.program_id(1)

    @pl.when(c == 0)
    def _init():
        carry_ref[...] = jnp.zeros_like(carry_ref)
        h_ref[...] = jnp.zeros_like(h_ref)

    xb = x_ref[0]
    xz = _dot(xb, winT_ref[...])
    xpart = xz[:, :DI]
    res_ = xz[:, DI:]

    ext = jnp.concatenate([carry_ref[...], xpart], axis=0)
    carry_ref[...] = xpart[C - (DCONV - 1):, :]
    conv_out = jnp.broadcast_to(cb_ref[...], (C, DI))
    for k in range(DCONV):
        conv_out = conv_out + cwT_ref[k:k + 1, :] * ext[k:k + C, :]
    u = _silu(conv_out)

    dt = _dot(u, xpdtT_ref[...])
    bm_ref[...] = _dot(u, xpBT_ref[...])
    cm_ref[...] = _dot(u, xpCT_ref[...])
    z = _dot(dt, dtT_ref[...]) + dtb_ref[...]
    delta = _softplus(z)
    dsc_ref[...] = delta
    u_ref[...] = u
    g_ref[...] = _silu(res_)
    ud_ref[...] = u * d_ref[...]

    AT = -jnp.exp(alogT_ref[...])
    eye = (lax.broadcasted_iota(jnp.int32, (DS, DS), 0)
           == lax.broadcasted_iota(jnp.int32, (DS, DS), 1)).astype(jnp.float32)

    def substep(t, h):
        drow = dsc_ref[pl.ds(t, 1), :]
        urow = u_ref[pl.ds(t, 1), :]
        brow = bm_ref[pl.ds(t, 1), :]
        crow = cm_ref[pl.ds(t, 1), :]
        E = jnp.exp(drow * AT)
        bcol = jnp.sum(jnp.broadcast_to(brow, (DS, DS)) * eye, axis=1,
                       keepdims=True)
        ccol = jnp.sum(jnp.broadcast_to(crow, (DS, DS)) * eye, axis=1,
                       keepdims=True)
        h = E * h + (drow * bcol) * urow
        yg_ref[pl.ds(t, 1), :] = jnp.sum(h * ccol, axis=0, keepdims=True)
        return h

    def step(i, h):
        t = i * 4
        for j in range(4):
            h = substep(t + j, h)
        return h

    h_ref[...] = lax.fori_loop(0, C // 4, step, h_ref[...])

    yg = (yg_ref[...] + ud_ref[...]) * g_ref[...]
    ctx = _dot(yg, opT_ref[...])
    lg_ref[0] = _dot(ctx, rwT_ref[...]) + rb_ref[...]


def kernel(x, in_proj_w, conv_w, conv_b, x_proj_w, dt_proj_w, dt_proj_b,
           A_log, D, out_proj_w, router_w, router_b):
    B, L, DM = x.shape
    DI, DCONV = conv_w.shape
    DTR = dt_proj_w.shape[1]
    DS = A_log.shape[1]
    NE = router_w.shape[0]
    C = min(256, L)
    NC = L // C

    winT = in_proj_w.T
    cwT = conv_w.T
    xpdtT = x_proj_w[:DTR].T
    xpBT = x_proj_w[DTR:DTR + DS].T
    xpCT = x_proj_w[DTR + DS:].T
    dtT = dt_proj_w.T
    alogT = A_log.T
    opT = out_proj_w.T
    rwT = router_w.T
    cb = conv_b.reshape(1, DI)
    dtb = dt_proj_b.reshape(1, DI)
    dd = D.reshape(1, DI)
    rb = router_b.reshape(1, NE)

    full = lambda a: pl.BlockSpec(a.shape, lambda b, c: (0,) * a.ndim)
    body = functools.partial(_mamba_body, C, DI, DS, DCONV, NE)
    logits = pl.pallas_call(
        body,
        grid=(B, NC),
        in_specs=[
            pl.BlockSpec((1, C, DM), lambda b, c: (b, c, 0)),
            full(winT), full(cwT), full(cb), full(xpdtT), full(xpBT),
            full(xpCT), full(dtT), full(dtb), full(alogT), full(dd),
            full(opT), full(rwT), full(rb),
        ],
        out_specs=pl.BlockSpec((1, C, NE), lambda b, c: (b, c, 0)),
        out_shape=jax.ShapeDtypeStruct((B, L, NE), jnp.float32),
        scratch_shapes=[
            pltpu.VMEM((DCONV - 1, DI), jnp.float32),
            pltpu.VMEM((DS, DI), jnp.float32),
            pltpu.VMEM((C, DI), jnp.float32),
            pltpu.VMEM((C, DI), jnp.float32),
            pltpu.VMEM((C, DI), jnp.float32),
            pltpu.VMEM((C, DI), jnp.float32),
            pltpu.VMEM((C, DS), jnp.float32),
            pltpu.VMEM((C, DS), jnp.float32),
            pltpu.VMEM((C, DI), jnp.float32),
        ],
    )(x, winT, cwT, cb, xpdtT, xpBT, xpCT, dtT, dtb, alogT, dd, opT, rwT, rb)

    logitsT = logits.reshape(B * L, NE).T
    wT, iT = _router_sc(logitsT)
    weights = wT.T.reshape(B, L, 2)
    idx = iT.T.reshape(B, L, 2)
    return (weights, idx)


# speedup vs baseline: 1.0866x; 1.0030x over previous
import functools

import jax
import jax.numpy as jnp
from jax import lax
from jax.experimental import pallas as pl
from jax.experimental.pallas import tpu as pltpu
from jax.experimental.pallas import tpu_sc as plsc

_PREC = lax.Precision.DEFAULT


def _dot(a, b):
    return jnp.dot(a, b, preferred_element_type=jnp.float32, precision=_PREC)

_SC_CORES = 2
_SC_SUBCORES = 16
_SC_LANES = 16


def _router_sc(logitsT):
    NE, TOK = logitsT.shape
    NW = _SC_CORES * _SC_SUBCORES
    TPW = TOK // NW
    mesh = plsc.VectorSubcoreMesh(
        core_axis_name="c", subcore_axis_name="s",
        num_cores=_SC_CORES, num_subcores=_SC_SUBCORES)

    @functools.partial(
        pl.kernel, mesh=mesh,
        out_type=[jax.ShapeDtypeStruct((2, TOK), jnp.float32),
                  jax.ShapeDtypeStruct((2, TOK), jnp.int32)],
        scratch_types=[pltpu.VMEM((NE, TPW), jnp.float32),
                       pltpu.VMEM((2, TPW), jnp.float32),
                       pltpu.VMEM((2, TPW), jnp.int32)],
    )
    def run(lg_hbm, w_hbm, i_hbm, lg_v, w_v, i_v):
        wid = lax.axis_index("s") * _SC_CORES + lax.axis_index("c")
        base = wid * TPW
        pltpu.sync_copy(lg_hbm.at[:, pl.ds(base, TPW)], lg_v)
        for g in range(TPW // _SC_LANES):
            sl = pl.ds(g * _SC_LANES, _SC_LANES)
            best1 = lg_v[0, sl]
            i1 = jnp.zeros((_SC_LANES,), jnp.int32)
            best2 = jnp.full((_SC_LANES,), -1e30, jnp.float32)
            i2 = jnp.zeros((_SC_LANES,), jnp.int32)
            for e in range(1, NE):
                v = lg_v[e, sl]
                gt1 = v > best1
                gt2 = v > best2
                i2 = jnp.where(gt1, i1, jnp.where(gt2, e, i2))
                best2 = jnp.where(gt1, best1, jnp.where(gt2, v, best2))
                i1 = jnp.where(gt1, e, i1)
                best1 = jnp.where(gt1, v, best1)
            w1 = 1.0 / (1.0 + jnp.exp(best2 - best1))
            w_v[0, sl] = w1
            w_v[1, sl] = 1.0 - w1
            i_v[0, sl] = i1
            i_v[1, sl] = i2
        pltpu.sync_copy(w_v, w_hbm.at[:, pl.ds(base, TPW)])
        pltpu.sync_copy(i_v, i_hbm.at[:, pl.ds(base, TPW)])

    return run(logitsT)


def _sigmoid(v):
    return lax.logistic(v)


def _silu(v):
    return v * _sigmoid(v)


def _softplus(v):
    return jnp.logaddexp(v, 0.0)


def _mamba_body(C, DI, DS, DCONV, NE,
                x_ref, winT_ref, cwT_ref, cb_ref, xpdtT_ref, xpBT_ref,
                xpCT_ref, dtT_ref, dtb_ref, alogT_ref, d_ref, opT_ref,
                rwT_ref, rb_ref,
                lg_ref,
                carry_ref, h_ref, dsc_ref, u_ref, g_ref, ud_ref,
                bm_ref, cm_ref, yg_ref):
    c = pl.program_id(1)

    @pl.when(c == 0)
    def _init():
        carry_ref[...] = jnp.zeros_like(carry_ref)
        h_ref[...] = jnp.zeros_like(h_ref)

    xb = x_ref[0]
    xz = _dot(xb, winT_ref[...])
    xpart = xz[:, :DI]
    res_ = xz[:, DI:]

    ext = jnp.concatenate([carry_ref[...], xpart], axis=0)
    carry_ref[...] = xpart[C - (DCONV - 1):, :]
    conv_out = jnp.broadcast_to(cb_ref[...], (C, DI))
    for k in range(DCONV):
        conv_out = conv_out + cwT_ref[k:k + 1, :] * ext[k:k + C, :]
    u = _silu(conv_out)

    dt = _dot(u, xpdtT_ref[...])
    bm_ref[...] = _dot(u, xpBT_ref[...])
    cm_ref[...] = _dot(u, xpCT_ref[...])
    z = _dot(dt, dtT_ref[...]) + dtb_ref[...]
    delta = _softplus(z)
    dsc_ref[...] = delta
    u_ref[...] = u
    g_ref[...] = _silu(res_)
    ud_ref[...] = u * d_ref[...]

    AT = -jnp.exp(alogT_ref[...])
    eye = (lax.broadcasted_iota(jnp.int32, (DS, DS), 0)
           == lax.broadcasted_iota(jnp.int32, (DS, DS), 1)).astype(jnp.float32)

    def substep(t, h):
        drow = dsc_ref[pl.ds(t, 1), :]
        urow = u_ref[pl.ds(t, 1), :]
        brow = bm_ref[pl.ds(t, 1), :]
        crow = cm_ref[pl.ds(t, 1), :]
        E = jnp.exp(drow * AT)
        bcol = jnp.sum(jnp.broadcast_to(brow, (DS, DS)) * eye, axis=1,
                       keepdims=True)
        ccol = jnp.sum(jnp.broadcast_to(crow, (DS, DS)) * eye, axis=1,
                       keepdims=True)
        h = E * h + (drow * bcol) * urow
        yg_ref[pl.ds(t, 1), :] = jnp.sum(h * ccol, axis=0, keepdims=True)
        return h

    def step(i, h):
        t = i * 8
        for j in range(8):
            h = substep(t + j, h)
        return h

    h_ref[...] = lax.fori_loop(0, C // 8, step, h_ref[...])

    yg = (yg_ref[...] + ud_ref[...]) * g_ref[...]
    ctx = _dot(yg, opT_ref[...])
    lg_ref[0] = _dot(ctx, rwT_ref[...]) + rb_ref[...]


def kernel(x, in_proj_w, conv_w, conv_b, x_proj_w, dt_proj_w, dt_proj_b,
           A_log, D, out_proj_w, router_w, router_b):
    B, L, DM = x.shape
    DI, DCONV = conv_w.shape
    DTR = dt_proj_w.shape[1]
    DS = A_log.shape[1]
    NE = router_w.shape[0]
    C = min(256, L)
    NC = L // C

    winT = in_proj_w.T
    cwT = conv_w.T
    xpdtT = x_proj_w[:DTR].T
    xpBT = x_proj_w[DTR:DTR + DS].T
    xpCT = x_proj_w[DTR + DS:].T
    dtT = dt_proj_w.T
    alogT = A_log.T
    opT = out_proj_w.T
    rwT = router_w.T
    cb = conv_b.reshape(1, DI)
    dtb = dt_proj_b.reshape(1, DI)
    dd = D.reshape(1, DI)
    rb = router_b.reshape(1, NE)

    full = lambda a: pl.BlockSpec(a.shape, lambda b, c: (0,) * a.ndim)
    body = functools.partial(_mamba_body, C, DI, DS, DCONV, NE)
    logits = pl.pallas_call(
        body,
        grid=(B, NC),
        in_specs=[
            pl.BlockSpec((1, C, DM), lambda b, c: (b, c, 0)),
            full(winT), full(cwT), full(cb), full(xpdtT), full(xpBT),
            full(xpCT), full(dtT), full(dtb), full(alogT), full(dd),
            full(opT), full(rwT), full(rb),
        ],
        out_specs=pl.BlockSpec((1, C, NE), lambda b, c: (b, c, 0)),
        out_shape=jax.ShapeDtypeStruct((B, L, NE), jnp.float32),
        scratch_shapes=[
            pltpu.VMEM((DCONV - 1, DI), jnp.float32),
            pltpu.VMEM((DS, DI), jnp.float32),
            pltpu.VMEM((C, DI), jnp.float32),
            pltpu.VMEM((C, DI), jnp.float32),
            pltpu.VMEM((C, DI), jnp.float32),
            pltpu.VMEM((C, DI), jnp.float32),
            pltpu.VMEM((C, DS), jnp.float32),
            pltpu.VMEM((C, DS), jnp.float32),
            pltpu.VMEM((C, DI), jnp.float32),
        ],
    )(x, winT, cwT, cb, xpdtT, xpBT, xpCT, dtT, dtb, alogT, dd, opT, rwT, rb)

    logitsT = logits.reshape(B * L, NE).T
    wT, iT = _router_sc(logitsT)
    weights = wT.T.reshape(B, L, 2)
    idx = iT.T.reshape(B, L, 2)
    return (weights, idx)
